# baseline (speedup 1.0000x reference)
import functools

import jax
import jax.numpy as jnp
from jax import lax
from jax.experimental import pallas as pl
from jax.experimental.pallas import tpu as pltpu
from jax.experimental.pallas import tpu_sc as plsc

EMBEDDING_DIM = 64
_NC, _NS = 2, 16
_NW = _NC * _NS
_CHUNK = 128
_L = 16


@functools.lru_cache(maxsize=None)
def _make_gather(B, V2, D):
    b_per_w = B // _NW
    n_chunks = b_per_w // _CHUNK
    mesh = plsc.VectorSubcoreMesh(core_axis_name="c", subcore_axis_name="s")

    @functools.partial(
        pl.kernel,
        mesh=mesh,
        out_type=jax.ShapeDtypeStruct((B, D), jnp.float32),
        scratch_types=[
            pltpu.VMEM((n_chunks, _CHUNK), jnp.int32),
            pltpu.VMEM((n_chunks, _CHUNK), jnp.int32),
            pltpu.VMEM((b_per_w, 2 * D), jnp.float32),
            pltpu.VMEM((b_per_w, D), jnp.float32),
            pltpu.SemaphoreType.DMA,
        ],
        compiler_params=pltpu.CompilerParams(
            use_tc_tiling_on_sc=False, needs_layout_passes=False),
    )
    def gather_kernel(idx_hbm, half_hbm, table_hbm, out_hbm,
                      idx_v, half_v, rows_v, out_v, sem):
        wid = lax.axis_index("s") * _NC + lax.axis_index("c")
        base = wid * b_per_w
        lanes = lax.iota(jnp.int32, _L)
        pltpu.sync_copy(idx_hbm.at[pl.ds(wid * n_chunks, n_chunks)], idx_v)
        pltpu.sync_copy(half_hbm.at[pl.ds(wid * n_chunks, n_chunks)], half_v)
        copies = []
        for j in range(n_chunks):
            copies.append(
                pltpu.async_copy(
                    table_hbm.at[idx_v.at[j]],
                    rows_v.at[pl.ds(j * _CHUNK, _CHUNK)],
                    sem,
                )
            )
        for c in copies:
            c.wait()

        def sel(g, carry):
            jvec = g * _L + lanes
            hvec = plsc.load_gather(
                half_v, [jvec // _CHUNK, jvec % _CHUNK]) * D

            def ext(r, c2):
                vals = plsc.load_gather(rows_v, [jvec, hvec + r])
                plsc.store_scatter(out_v, [jvec, lanes * 0 + r], vals)
                return c2

            lax.fori_loop(0, D, ext, 0)
            return carry

        lax.fori_loop(0, b_per_w // _L, sel, 0)
        pltpu.sync_copy(out_v, out_hbm.at[pl.ds(base, b_per_w)])

    return gather_kernel


def kernel(entities, entity_embeddings):
    B = entities.shape[0]
    V, D = entity_embeddings.shape
    wide = entity_embeddings.reshape(V // 2, 2 * D)
    idx = entities.astype(jnp.int32)
    idx2d = (idx >> 1).reshape(B // _CHUNK, _CHUNK)
    half2d = (idx & 1).reshape(B // _CHUNK, _CHUNK)
    out = _make_gather(B, V // 2, D)(idx2d, half2d, wide)
    return out.reshape(-1, D)
